# baseline (speedup 1.0000x reference)
import functools

import jax
import jax.numpy as jnp
from jax import lax
from jax.experimental import pallas as pl
from jax.experimental.pallas import tpu as pltpu
from jax.experimental.pallas import tpu_sc as plsc

_VP = 128
_B = 5088
_NC, _NS = 2, 16
_NW = _NC * _NS
_C = 104
_NSC = _NW * _C


def _tc_body(xt_ref, r_ref, t_ref, o_ref):
    _, f, b = xt_ref.shape
    xt = xt_ref[0]
    riota = r_ref[...]
    parts = []
    for i in range(f):
        parts.append(
            jnp.where(xt[i][None, :] == riota, jnp.bfloat16(1), jnp.bfloat16(0))
        )
    oht = jnp.concatenate(parts, axis=0)
    o_ref[...] = jax.lax.dot_general(
        oht, t_ref[...],
        dimension_numbers=(((0,), (0,)), ((), ())),
        preferred_element_type=jnp.float32,
    )


def _sc_encode(flat_idx_wf, table_flat, s, h):
    nw, chunks_per_w, f, c = flat_idx_wf.shape
    mesh = plsc.VectorSubcoreMesh(core_axis_name="c", subcore_axis_name="s")

    @functools.partial(
        pl.kernel,
        mesh=mesh,
        out_type=jax.ShapeDtypeStruct((s, h), jnp.float32),
        scratch_types=[
            pltpu.VMEM((f, _C), jnp.int32),
            pltpu.VMEM((2, _C, 256), jnp.float32),
            pltpu.VMEM((_C, 256), jnp.float32),
            pltpu.SemaphoreType.DMA,
            pltpu.SemaphoreType.DMA,
        ],
    )
    def k(tab_hbm, idx_hbm, out_hbm, idx_v, tmp_v, acc_v, sem0, sem1):
        cid = lax.axis_index("c")
        sid = lax.axis_index("s")
        wid = sid * _NC + cid
        sems = (sem0, sem1)

        @pl.loop(0, chunks_per_w)
        def _(g):
            base = (wid * chunks_per_w + g) * _C
            pltpu.sync_copy(idx_hbm.at[wid, g], idx_v)
            pltpu.async_copy(tab_hbm.at[idx_v.at[0]], acc_v, sem0).wait()
            pltpu.async_copy(tab_hbm.at[idx_v.at[1]], tmp_v.at[0], sem0)
            for i in range(1, f):
                cur = (i - 1) % 2
                if i + 1 < f:
                    pltpu.async_copy(
                        tab_hbm.at[idx_v.at[i + 1]], tmp_v.at[i % 2],
                        sems[i % 2],
                    )
                pltpu.make_async_copy(
                    tab_hbm.at[idx_v.at[i]], tmp_v.at[cur], sems[cur]
                ).wait()

                @plsc.parallel_loop(0, _C, step=1, unroll=4)
                def _(r):
                    @plsc.parallel_loop(0, 256, step=16, unroll=4)
                    def _(c):
                        plsc.addupdate(
                            acc_v.at[r, pl.ds(c, 16)],
                            tmp_v[cur, r, pl.ds(c, 16)],
                        )

            pltpu.sync_copy(acc_v, out_hbm.at[pl.ds(base, _C)])

    return k(table_flat, flat_idx_wf)


def kernel(x, tables):
    if x.ndim == 1:
        x = x[:, None]
    n, f = x.shape
    nf, v, h = tables.shape
    n_tc = n - _NSC
    nb = n_tc // _B

    xsc = x[n_tc:].astype(jnp.int32)
    flat = xsc + (jnp.arange(f, dtype=jnp.int32) * v)[None, :]
    flat_wf = flat.reshape(_NW, _NSC // (_NW * _C), _C, f).transpose(0, 1, 3, 2)
    out_sc = _sc_encode(flat_wf, tables.reshape(nf * v, h), _NSC, h)

    xt = x[:n_tc].T.astype(jnp.bfloat16).reshape(f, nb, _B).transpose(1, 0, 2)
    riota = jnp.broadcast_to(
        jnp.arange(_VP, dtype=jnp.bfloat16)[:, None], (_VP, _B))
    tp = jnp.pad(tables, ((0, 0), (0, _VP - v), (0, 0)))
    tp = tp.astype(jnp.bfloat16).reshape(f * _VP, h)
    out_tc = pl.pallas_call(
        _tc_body,
        grid=(nb,),
        in_specs=[
            pl.BlockSpec((1, f, _B), lambda i: (i, 0, 0)),
            pl.BlockSpec((_VP, _B), lambda i: (0, 0)),
            pl.BlockSpec((f * _VP, h), lambda i: (0, 0)),
        ],
        out_specs=pl.BlockSpec((_B, h), lambda i: (i, 0)),
        out_shape=jax.ShapeDtypeStruct((n, h), jnp.float32),
    )(xt, riota, tp)

    return jax.lax.dynamic_update_slice(out_tc, out_sc, (n_tc, 0))


# speedup vs baseline: 1.0324x; 1.0324x over previous
import functools

import jax
import jax.numpy as jnp
from jax import lax
from jax.experimental import pallas as pl
from jax.experimental.pallas import tpu as pltpu
from jax.experimental.pallas import tpu_sc as plsc

_VP = 128
_B = 7088
_NC, _NS = 2, 16
_NW = _NC * _NS
_C = 24
_NSC = _NW * _C


def _tc_body(xt_ref, r_ref, t_ref, o_ref):
    _, f, b = xt_ref.shape
    xt = xt_ref[0]
    riota = r_ref[...]
    parts = []
    for i in range(f):
        parts.append(
            jnp.where(xt[i][None, :] == riota, jnp.bfloat16(1), jnp.bfloat16(0))
        )
    oht = jnp.concatenate(parts, axis=0)
    o_ref[...] = jax.lax.dot_general(
        oht, t_ref[...],
        dimension_numbers=(((0,), (0,)), ((), ())),
        preferred_element_type=jnp.float32,
    )


def _sc_encode(flat_idx_wf, table_flat, s, h):
    nw, chunks_per_w, f, c = flat_idx_wf.shape
    mesh = plsc.VectorSubcoreMesh(core_axis_name="c", subcore_axis_name="s")

    @functools.partial(
        pl.kernel,
        mesh=mesh,
        out_type=jax.ShapeDtypeStruct((s, h), jnp.float32),
        scratch_types=[
            pltpu.VMEM((f, _C), jnp.int32),
            pltpu.VMEM((2, _C, 256), jnp.float32),
            pltpu.VMEM((_C, 256), jnp.float32),
            pltpu.SemaphoreType.DMA,
            pltpu.SemaphoreType.DMA,
        ],
    )
    def k(tab_hbm, idx_hbm, out_hbm, idx_v, tmp_v, acc_v, sem0, sem1):
        cid = lax.axis_index("c")
        sid = lax.axis_index("s")
        wid = sid * _NC + cid
        sems = (sem0, sem1)

        @pl.loop(0, chunks_per_w)
        def _(g):
            base = (wid * chunks_per_w + g) * _C
            pltpu.sync_copy(idx_hbm.at[wid, g], idx_v)
            pltpu.async_copy(tab_hbm.at[idx_v.at[0]], acc_v, sem0).wait()
            pltpu.async_copy(tab_hbm.at[idx_v.at[1]], tmp_v.at[0], sem0)
            for i in range(1, f):
                cur = (i - 1) % 2
                if i + 1 < f:
                    pltpu.async_copy(
                        tab_hbm.at[idx_v.at[i + 1]], tmp_v.at[i % 2],
                        sems[i % 2],
                    )
                pltpu.make_async_copy(
                    tab_hbm.at[idx_v.at[i]], tmp_v.at[cur], sems[cur]
                ).wait()

                @plsc.parallel_loop(0, _C, step=1, unroll=4)
                def _(r):
                    @plsc.parallel_loop(0, 256, step=16, unroll=4)
                    def _(c):
                        plsc.addupdate(
                            acc_v.at[r, pl.ds(c, 16)],
                            tmp_v[cur, r, pl.ds(c, 16)],
                        )

            pltpu.sync_copy(acc_v, out_hbm.at[pl.ds(base, _C)])

    return k(table_flat, flat_idx_wf)


def kernel(x, tables):
    if x.ndim == 1:
        x = x[:, None]
    n, f = x.shape
    nf, v, h = tables.shape
    n_tc = n - _NSC
    nb = n_tc // _B

    xsc = x[n_tc:].astype(jnp.int32)
    flat = xsc + (jnp.arange(f, dtype=jnp.int32) * v)[None, :]
    flat_wf = flat.reshape(_NW, _NSC // (_NW * _C), _C, f).transpose(0, 1, 3, 2)
    out_sc = _sc_encode(flat_wf, tables.reshape(nf * v, h), _NSC, h)

    xt = x[:n_tc].T.astype(jnp.bfloat16).reshape(f, nb, _B).transpose(1, 0, 2)
    riota = jnp.broadcast_to(
        jnp.arange(_VP, dtype=jnp.bfloat16)[:, None], (_VP, _B))
    tp = jnp.pad(tables, ((0, 0), (0, _VP - v), (0, 0)))
    tp = tp.astype(jnp.bfloat16).reshape(f * _VP, h)
    out_tc = pl.pallas_call(
        _tc_body,
        grid=(nb,),
        in_specs=[
            pl.BlockSpec((1, f, _B), lambda i: (i, 0, 0)),
            pl.BlockSpec((_VP, _B), lambda i: (0, 0)),
            pl.BlockSpec((f * _VP, h), lambda i: (0, 0)),
        ],
        out_specs=pl.BlockSpec((_B, h), lambda i: (i, 0)),
        out_shape=jax.ShapeDtypeStruct((n, h), jnp.float32),
    )(xt, riota, tp)

    return jax.lax.dynamic_update_slice(out_tc, out_sc, (n_tc, 0))
